# baseline (speedup 1.0000x reference)
import functools

import jax
import jax.numpy as jnp
from jax import lax
from jax.experimental import pallas as pl
from jax.experimental.pallas import tpu as pltpu
from jax.experimental.pallas import tpu_sc as plsc

VOCAB = 1000000
EMB = 64
HID = 128
NCLS = 50
B = 4096
L = 200

NC = 2
NS = 16
NW = NC * NS
ROWS_PER_W = B // NW
CH0, CH1 = 128, L - 128


def _sc_pool_kernel(x_hbm, emb_hbm, out_hbm, idx_v, buf, out_v, sem):
    wid = lax.axis_index("s") * NC + lax.axis_index("c")
    base = wid * ROWS_PER_W
    pltpu.sync_copy(x_hbm.at[pl.ds(base, ROWS_PER_W)], idx_v)

    def body(r, carry):
        d0 = pltpu.async_copy(
            emb_hbm.at[idx_v.at[r, pl.ds(0, CH0)]], buf.at[pl.ds(0, CH0)], sem)
        d1 = pltpu.async_copy(
            emb_hbm.at[idx_v.at[r, pl.ds(CH0, CH1)]], buf.at[pl.ds(CH0, CH1)], sem)
        d0.wait()
        d1.wait()
        for c in range(EMB // 16):
            acc = buf[0, pl.ds(c * 16, 16)]
            for l in range(1, L):
                acc = acc + buf[l, pl.ds(c * 16, 16)]
            out_v[r, pl.ds(c * 16, 16)] = acc
        return carry

    lax.fori_loop(0, ROWS_PER_W, body, 0)
    pltpu.sync_copy(out_v, out_hbm.at[pl.ds(base, ROWS_PER_W)])


_sc_pool = functools.partial(
    pl.kernel,
    mesh=plsc.VectorSubcoreMesh(core_axis_name="c", subcore_axis_name="s"),
    compiler_params=pltpu.CompilerParams(use_tc_tiling_on_sc=False),
    out_type=jax.ShapeDtypeStruct((B, EMB), jnp.float32),
    scratch_types=[
        pltpu.VMEM((ROWS_PER_W, L), jnp.int32),
        pltpu.VMEM((L, EMB), jnp.float32),
        pltpu.VMEM((ROWS_PER_W, EMB), jnp.float32),
        pltpu.SemaphoreType.DMA,
    ],
)(_sc_pool_kernel)


def _mlp_kernel(p_ref, w1_ref, b1_ref, w2_ref, b2_ref, o_ref):
    p = p_ref[...] * (1.0 / L)
    h = jnp.dot(p, w1_ref[...], preferred_element_type=jnp.float32) + b1_ref[...]
    h = jnp.maximum(h, 0.0)
    o_ref[...] = jnp.dot(h, w2_ref[...], preferred_element_type=jnp.float32) + b2_ref[...]


def kernel(x, emb, W1, b1, W2, b2):
    pooled_sum = _sc_pool(x, emb)
    out = pl.pallas_call(
        _mlp_kernel,
        out_shape=jax.ShapeDtypeStruct((B, NCLS), jnp.float32),
    )(pooled_sum, W1, b1.reshape(1, HID), W2, b2.reshape(1, NCLS))
    return out


# speedup vs baseline: 1.5272x; 1.5272x over previous
import functools

import jax
import jax.numpy as jnp
from jax import lax
from jax.experimental import pallas as pl
from jax.experimental.pallas import tpu as pltpu
from jax.experimental.pallas import tpu_sc as plsc

VOCAB = 1000000
EMB = 64
HID = 128
NCLS = 50
B = 4096
L = 200

NC = 2
NS = 16
NW = NC * NS
ROWS_PER_W = B // NW
CH0, CH1 = 128, L - 128
NBUF = 4


def _sc_pool_kernel(x_hbm, emb_hbm, out_hbm,
                    idx_v, b0, b1, b2, b3, out_v, s0, s1, s2, s3):
    bufs = (b0, b1, b2, b3)
    sems = (s0, s1, s2, s3)
    wid = lax.axis_index("s") * NC + lax.axis_index("c")
    base = wid * ROWS_PER_W
    pltpu.sync_copy(x_hbm.at[pl.ds(base, ROWS_PER_W)], idx_v)

    def fire(r, buf, sem):
        pltpu.async_copy(
            emb_hbm.at[idx_v.at[r, pl.ds(0, CH0)]], buf.at[pl.ds(0, CH0)], sem)
        pltpu.async_copy(
            emb_hbm.at[idx_v.at[r, pl.ds(CH0, CH1)]], buf.at[pl.ds(CH0, CH1)], sem)

    def drain(buf, sem):
        pltpu.make_async_copy(emb_hbm.at[pl.ds(0, L)], buf, sem).wait()

    def reduce_row(buf, r):
        def rbody(lo, accs):
            a0, a1, a2, a3 = accs
            for j in range(8):
                l = lo * 8 + j
                a0 = a0 + buf[l, pl.ds(0, 16)]
                a1 = a1 + buf[l, pl.ds(16, 16)]
                a2 = a2 + buf[l, pl.ds(32, 16)]
                a3 = a3 + buf[l, pl.ds(48, 16)]
            return a0, a1, a2, a3
        z = jnp.zeros((16,), jnp.float32)
        a0, a1, a2, a3 = lax.fori_loop(0, L // 8, rbody, (z, z, z, z))
        out_v[r, pl.ds(0, 16)] = a0
        out_v[r, pl.ds(16, 16)] = a1
        out_v[r, pl.ds(32, 16)] = a2
        out_v[r, pl.ds(48, 16)] = a3

    for b in range(NBUF):
        fire(b, bufs[b], sems[b])

    def gbody(g, carry):
        for b in range(NBUF):
            r = g * NBUF + b
            drain(bufs[b], sems[b])
            reduce_row(bufs[b], r)
            nxt = r + NBUF

            @pl.when(nxt < ROWS_PER_W)
            def _():
                fire(nxt, bufs[b], sems[b])
        return carry

    lax.fori_loop(0, ROWS_PER_W // NBUF, gbody, 0)
    pltpu.sync_copy(out_v, out_hbm.at[pl.ds(base, ROWS_PER_W)])


_sc_pool = functools.partial(
    pl.kernel,
    mesh=plsc.VectorSubcoreMesh(core_axis_name="c", subcore_axis_name="s"),
    compiler_params=pltpu.CompilerParams(use_tc_tiling_on_sc=False),
    out_type=jax.ShapeDtypeStruct((B, EMB), jnp.float32),
    scratch_types=[
        pltpu.VMEM((ROWS_PER_W, L), jnp.int32),
        pltpu.VMEM((L, EMB), jnp.float32),
        pltpu.VMEM((L, EMB), jnp.float32),
        pltpu.VMEM((L, EMB), jnp.float32),
        pltpu.VMEM((L, EMB), jnp.float32),
        pltpu.VMEM((ROWS_PER_W, EMB), jnp.float32),
        pltpu.SemaphoreType.DMA,
        pltpu.SemaphoreType.DMA,
        pltpu.SemaphoreType.DMA,
        pltpu.SemaphoreType.DMA,
    ],
)(_sc_pool_kernel)


def _mlp_kernel(p_ref, w1_ref, b1_ref, w2_ref, b2_ref, o_ref):
    p = p_ref[...] * (1.0 / L)
    h = jnp.dot(p, w1_ref[...], preferred_element_type=jnp.float32) + b1_ref[...]
    h = jnp.maximum(h, 0.0)
    o_ref[...] = jnp.dot(h, w2_ref[...], preferred_element_type=jnp.float32) + b2_ref[...]


def kernel(x, emb, W1, b1, W2, b2):
    pooled_sum = _sc_pool(x, emb)
    out = pl.pallas_call(
        _mlp_kernel,
        out_shape=jax.ShapeDtypeStruct((B, NCLS), jnp.float32),
    )(pooled_sum, W1, b1.reshape(1, HID), W2, b2.reshape(1, NCLS))
    return out
